# baseline (speedup 1.0000x reference)
import functools

import jax
import jax.numpy as jnp
from jax import lax
from jax.experimental import pallas as pl
from jax.experimental.pallas import tpu as pltpu
from jax.experimental.pallas import tpu_sc as plsc

B = 1024
D = 16
NUM_ENT = 1000000
NUM_TYPE = 1000
NUM_CORES = 2
NUM_SUBCORES = 16
NW = NUM_CORES * NUM_SUBCORES
B_PER_W = B // NW
SLAB = 128
GRP = 16


def _sc_gather(ent, ent_type, ee_t, te_t, ep_t, tp_t):
    mesh = plsc.VectorSubcoreMesh(core_axis_name="c", subcore_axis_name="s")
    row_blk_t = jax.ShapeDtypeStruct((NW, B_PER_W, D), jnp.float32)

    @functools.partial(
        pl.kernel,
        mesh=mesh,
        compiler_params=pltpu.CompilerParams(needs_layout_passes=False),
        out_type=[row_blk_t, row_blk_t, row_blk_t, row_blk_t],
        scratch_types=[
            pltpu.VMEM((B_PER_W,), jnp.int32),
            pltpu.VMEM((B_PER_W,), jnp.int32),
            pltpu.VMEM((GRP, D, SLAB), jnp.float32),
            pltpu.VMEM((GRP, D, SLAB), jnp.float32),
            pltpu.VMEM((D, NUM_TYPE), jnp.float32),
            pltpu.VMEM((D, NUM_TYPE), jnp.float32),
            pltpu.VMEM((B_PER_W, D), jnp.float32),
            pltpu.VMEM((B_PER_W, D), jnp.float32),
            pltpu.VMEM((B_PER_W, D), jnp.float32),
            pltpu.VMEM((B_PER_W, D), jnp.float32),
            pltpu.SemaphoreType.DMA,
            pltpu.SemaphoreType.DMA,
            pltpu.SemaphoreType.DMA,
            pltpu.SemaphoreType.DMA,
        ],
    )
    def gather_kernel(ent_hbm, etype_hbm, ee_hbm, te_hbm, ep_hbm, tp_hbm,
                      e_out, t_out, pe_out, pt_out,
                      idx_e, idx_t, slab0, slab1, ty_e, ty_p,
                      e_v, t_v, pe_v, pt_v, sem0, sem1, sem_ty, sem_out):
        wid = lax.axis_index("s") * NUM_CORES + lax.axis_index("c")
        base = wid * B_PER_W
        sl = pl.ds(base, B_PER_W)
        ci_e = pltpu.async_copy(ent_hbm.at[sl], idx_e, sem_out)
        ci_t = pltpu.async_copy(etype_hbm.at[sl], idx_t, sem_out)
        ct_e = pltpu.async_copy(te_hbm, ty_e, sem_ty)
        ct_p = pltpu.async_copy(tp_hbm, ty_p, sem_ty)
        ci_e.wait()
        grp16 = lax.iota(jnp.int32, GRP)
        rings = (slab0, slab1)
        sems = (sem0, sem1)

        def fire(table_hbm, g, ring):
            ve = idx_e[pl.ds(g * GRP, GRP)]
            cps = []
            for k in range(GRP):
                ie = ve[k]
                col = pl.multiple_of(ie - lax.rem(ie, SLAB), SLAB)
                cps.append(pltpu.async_copy(
                    table_hbm.at[:, pl.ds(col, SLAB)],
                    rings[ring].at[k], sems[ring]))
            return cps

        def extract(dst_v, g, ring):
            ve = idx_e[pl.ds(g * GRP, GRP)]
            lane_v = ve % SLAB
            row_v = grp16 + g * GRP
            for d in range(D):
                d_v = jnp.full((GRP,), d, jnp.int32)
                vec = plsc.load_gather(rings[ring], [grp16, d_v, lane_v])
                plsc.store_scatter(dst_v, [row_v, d_v], vec)

        rounds = [(e_v, 0), (e_v, 1), (pe_v, 0), (pe_v, 1)]
        tables = [ee_hbm, ee_hbm, ep_hbm, ep_hbm]
        inflight = [None, None]
        for i, ((dst, g), tbl) in enumerate(zip(rounds, tables)):
            ring = i % 2
            inflight[ring] = fire(tbl, g, ring)
            if i == 0:
                ci_t.wait()
                ct_e.wait()
                ct_p.wait()
                for g2 in range(B_PER_W // GRP):
                    gsl = pl.ds(g2 * GRP, GRP)
                    vt = idx_t[gsl]
                    row_v2 = grp16 + g2 * GRP
                    for d in range(D):
                        d_v = jnp.full((GRP,), d, jnp.int32)
                        vec_t = plsc.load_gather(ty_e, [d_v, vt])
                        plsc.store_scatter(t_v, [row_v2, d_v], vec_t)
                        vec_p = plsc.load_gather(ty_p, [d_v, vt])
                        plsc.store_scatter(pt_v, [row_v2, d_v], vec_p)
                co_t = pltpu.async_copy(t_v, t_out.at[wid], sem_out)
                co_pt = pltpu.async_copy(pt_v, pt_out.at[wid], sem_out)
            if i >= 1:
                for c in inflight[(i - 1) % 2]:
                    c.wait()
                dst_prev, g_prev = rounds[i - 1]
                extract(dst_prev, g_prev, (i - 1) % 2)
                if rounds[i - 1][0] is e_v and g_prev == 1:
                    co_e = pltpu.async_copy(e_v, e_out.at[wid], sem_out)
        for c in inflight[1]:
            c.wait()
        extract(pe_v, 1, 1)
        co_pe = pltpu.async_copy(pe_v, pe_out.at[wid], sem_out)
        co_t.wait()
        co_pt.wait()
        co_e.wait()
        co_pe.wait()

    return gather_kernel(ent, ent_type, ee_t, te_t, ep_t, tp_t)


BI = 512


def _score_body(e_ref, t_ref, pe_ref, pt_ref, out_ref):
    bf = jnp.bfloat16
    e = e_ref[...].reshape(BI, D).astype(bf)
    t = t_ref[...].reshape(BI, D).astype(bf)
    pe = pe_ref[...].reshape(B, D).astype(bf)
    pt = pt_ref[...].reshape(B, D).astype(bf)
    eye = jax.lax.broadcasted_iota(jnp.int32, (D, D), 0)
    eye = (eye == jax.lax.broadcasted_iota(jnp.int32, (D, D), 1)).astype(bf)
    pet = jax.lax.dot_general(eye, pe, (((1,), (1,)), ((), ())),
                              preferred_element_type=jnp.float32
                              ).astype(bf)
    ptt = jax.lax.dot_general(eye, pt, (((1,), (1,)), ((), ())),
                              preferred_element_type=jnp.float32
                              ).astype(bf)
    a = jax.lax.dot_general(e, pet, (((1,), (0,)), ((), ())),
                            preferred_element_type=jnp.float32
                            ).astype(jnp.bfloat16)
    c = jax.lax.dot_general(t, ptt, (((1,), (0,)), ((), ())),
                            preferred_element_type=jnp.float32
                            ).astype(jnp.bfloat16)
    diff = e - t
    out = jnp.zeros((BI, B), jnp.float32)
    for h in range(4):
        acc = jnp.zeros((BI, B), jnp.bfloat16)
        for d in range(4 * h, 4 * h + 4):
            term = diff[:, d:d + 1] + a * pet[d:d + 1, :] - c * ptt[d:d + 1, :]
            acc = acc + jnp.abs(term)
        out = out + acc.astype(jnp.float32)
    out_ref[...] = out


WPB = BI // B_PER_W


def _tc_score(e_b, t_b, pe_b, pt_b):
    return pl.pallas_call(
        _score_body,
        grid=(B // BI,),
        in_specs=[
            pl.BlockSpec((WPB, B_PER_W, D), lambda i: (i, 0, 0)),
            pl.BlockSpec((WPB, B_PER_W, D), lambda i: (i, 0, 0)),
            pl.BlockSpec((NW, B_PER_W, D), lambda i: (0, 0, 0)),
            pl.BlockSpec((NW, B_PER_W, D), lambda i: (0, 0, 0)),
        ],
        out_specs=pl.BlockSpec((BI, B), lambda i: (i, 0)),
        out_shape=jax.ShapeDtypeStruct((B, B), jnp.float32),
    )(e_b, t_b, pe_b, pt_b)


def kernel(ent, ent_type, ent_emb, type_emb, ent_proj, type_proj):
    e_b, t_b, pe_b, pt_b = _sc_gather(
        ent, ent_type,
        ent_emb.T, type_emb.T, ent_proj.T, type_proj.T)
    return _tc_score(e_b, t_b, pe_b, pt_b)


# speedup vs baseline: 1.0092x; 1.0092x over previous
import functools

import jax
import jax.numpy as jnp
from jax import lax
from jax.experimental import pallas as pl
from jax.experimental.pallas import tpu as pltpu
from jax.experimental.pallas import tpu_sc as plsc

B = 1024
D = 16
NUM_ENT = 1000000
NUM_TYPE = 1000
NUM_CORES = 2
NUM_SUBCORES = 16
NW = NUM_CORES * NUM_SUBCORES
B_PER_W = B // NW
SLAB = 128
GRP = 16


def _sc_gather(ent, ent_type, ee_t, te_t, ep_t, tp_t):
    mesh = plsc.VectorSubcoreMesh(core_axis_name="c", subcore_axis_name="s")
    row_blk_t = jax.ShapeDtypeStruct((NW, B_PER_W, D), jnp.float32)

    @functools.partial(
        pl.kernel,
        mesh=mesh,
        compiler_params=pltpu.CompilerParams(needs_layout_passes=False),
        out_type=[row_blk_t, row_blk_t, row_blk_t, row_blk_t],
        scratch_types=[
            pltpu.VMEM((B_PER_W,), jnp.int32),
            pltpu.VMEM((B_PER_W,), jnp.int32),
            pltpu.VMEM((GRP, D, SLAB), jnp.float32),
            pltpu.VMEM((GRP, D, SLAB), jnp.float32),
            pltpu.VMEM((D, NUM_TYPE), jnp.float32),
            pltpu.VMEM((D, NUM_TYPE), jnp.float32),
            pltpu.VMEM((B_PER_W, D), jnp.float32),
            pltpu.VMEM((B_PER_W, D), jnp.float32),
            pltpu.VMEM((B_PER_W, D), jnp.float32),
            pltpu.VMEM((B_PER_W, D), jnp.float32),
            pltpu.SemaphoreType.DMA,
            pltpu.SemaphoreType.DMA,
            pltpu.SemaphoreType.DMA,
            pltpu.SemaphoreType.DMA,
        ],
    )
    def gather_kernel(ent_hbm, etype_hbm, ee_hbm, te_hbm, ep_hbm, tp_hbm,
                      e_out, t_out, pe_out, pt_out,
                      idx_e, idx_t, slab0, slab1, ty_e, ty_p,
                      e_v, t_v, pe_v, pt_v, sem0, sem1, sem_ty, sem_out):
        wid = lax.axis_index("s") * NUM_CORES + lax.axis_index("c")
        base = wid * B_PER_W
        sl = pl.ds(base, B_PER_W)
        ci_e = pltpu.async_copy(ent_hbm.at[sl], idx_e, sem_out)
        ci_t = pltpu.async_copy(etype_hbm.at[sl], idx_t, sem_out)
        ct_e = pltpu.async_copy(te_hbm, ty_e, sem_ty)
        ct_p = pltpu.async_copy(tp_hbm, ty_p, sem_ty)
        ci_e.wait()
        ci_t.wait()
        grp16 = lax.iota(jnp.int32, GRP)
        rings = (slab0, slab1)
        sems = (sem0, sem1)

        def fire(table_hbm, g, ring):
            ve = idx_e[pl.ds(g * GRP, GRP)]
            cps = []
            for k in range(GRP):
                ie = ve[k]
                col = pl.multiple_of(ie - lax.rem(ie, SLAB), SLAB)
                cps.append(pltpu.async_copy(
                    table_hbm.at[:, pl.ds(col, SLAB)],
                    rings[ring].at[k], sems[ring]))
            return cps

        def extract(dst_v, g, ring):
            ve = idx_e[pl.ds(g * GRP, GRP)]
            lane_v = ve % SLAB
            row_v = grp16 + g * GRP
            for d in range(D):
                d_v = jnp.full((GRP,), d, jnp.int32)
                vec = plsc.load_gather(rings[ring], [grp16, d_v, lane_v])
                plsc.store_scatter(dst_v, [row_v, d_v], vec)

        rounds = [(e_v, 0), (e_v, 1), (pe_v, 0), (pe_v, 1)]
        tables = [ee_hbm, ee_hbm, ep_hbm, ep_hbm]
        inflight = [None, None]
        for i, ((dst, g), tbl) in enumerate(zip(rounds, tables)):
            ring = i % 2
            inflight[ring] = fire(tbl, g, ring)
            if i == 0:
                ct_e.wait()
                ct_p.wait()
                for g2 in range(B_PER_W // GRP):
                    gsl = pl.ds(g2 * GRP, GRP)
                    vt = idx_t[gsl]
                    row_v2 = grp16 + g2 * GRP
                    for d in range(D):
                        d_v = jnp.full((GRP,), d, jnp.int32)
                        vec_t = plsc.load_gather(ty_e, [d_v, vt])
                        plsc.store_scatter(t_v, [row_v2, d_v], vec_t)
                        vec_p = plsc.load_gather(ty_p, [d_v, vt])
                        plsc.store_scatter(pt_v, [row_v2, d_v], vec_p)
                co_t = pltpu.async_copy(t_v, t_out.at[wid], sem_out)
                co_pt = pltpu.async_copy(pt_v, pt_out.at[wid], sem_out)
            if i >= 1:
                for c in inflight[(i - 1) % 2]:
                    c.wait()
                dst_prev, g_prev = rounds[i - 1]
                extract(dst_prev, g_prev, (i - 1) % 2)
                if rounds[i - 1][0] is e_v and g_prev == 1:
                    co_e = pltpu.async_copy(e_v, e_out.at[wid], sem_out)
        for c in inflight[1]:
            c.wait()
        extract(pe_v, 1, 1)
        co_pe = pltpu.async_copy(pe_v, pe_out.at[wid], sem_out)
        co_t.wait()
        co_pt.wait()
        co_e.wait()
        co_pe.wait()

    return gather_kernel(ent, ent_type, ee_t, te_t, ep_t, tp_t)


BI = 512


def _score_body(e_ref, t_ref, pe_ref, pt_ref, out_ref):
    bf = jnp.bfloat16
    e = e_ref[...].reshape(BI, D).astype(bf)
    t = t_ref[...].reshape(BI, D).astype(bf)
    pe = pe_ref[...].reshape(B, D).astype(bf)
    pt = pt_ref[...].reshape(B, D).astype(bf)
    eye = jax.lax.broadcasted_iota(jnp.int32, (D, D), 0)
    eye = (eye == jax.lax.broadcasted_iota(jnp.int32, (D, D), 1)).astype(bf)
    pet = jax.lax.dot_general(eye, pe, (((1,), (1,)), ((), ())),
                              preferred_element_type=jnp.float32
                              ).astype(bf)
    ptt = jax.lax.dot_general(eye, pt, (((1,), (1,)), ((), ())),
                              preferred_element_type=jnp.float32
                              ).astype(bf)
    a = jax.lax.dot_general(e, pet, (((1,), (0,)), ((), ())),
                            preferred_element_type=jnp.float32
                            ).astype(jnp.bfloat16)
    c = jax.lax.dot_general(t, ptt, (((1,), (0,)), ((), ())),
                            preferred_element_type=jnp.float32
                            ).astype(jnp.bfloat16)
    diff = e - t
    out = jnp.zeros((BI, B), jnp.float32)
    for h in range(4):
        acc = jnp.zeros((BI, B), jnp.bfloat16)
        for d in range(4 * h, 4 * h + 4):
            term = diff[:, d:d + 1] + a * pet[d:d + 1, :] - c * ptt[d:d + 1, :]
            acc = acc + jnp.abs(term)
        out = out + acc.astype(jnp.float32)
    out_ref[...] = out


WPB = BI // B_PER_W


def _tc_score(e_b, t_b, pe_b, pt_b):
    return pl.pallas_call(
        _score_body,
        grid=(B // BI,),
        in_specs=[
            pl.BlockSpec((WPB, B_PER_W, D), lambda i: (i, 0, 0)),
            pl.BlockSpec((WPB, B_PER_W, D), lambda i: (i, 0, 0)),
            pl.BlockSpec((NW, B_PER_W, D), lambda i: (0, 0, 0)),
            pl.BlockSpec((NW, B_PER_W, D), lambda i: (0, 0, 0)),
        ],
        out_specs=pl.BlockSpec((BI, B), lambda i: (i, 0)),
        out_shape=jax.ShapeDtypeStruct((B, B), jnp.float32),
    )(e_b, t_b, pe_b, pt_b)


def kernel(ent, ent_type, ent_emb, type_emb, ent_proj, type_proj):
    e_b, t_b, pe_b, pt_b = _sc_gather(
        ent, ent_type,
        ent_emb.T, type_emb.T, ent_proj.T, type_proj.T)
    return _tc_score(e_b, t_b, pe_b, pt_b)


# speedup vs baseline: 1.0143x; 1.0050x over previous
import functools

import jax
import jax.numpy as jnp
from jax import lax
from jax.experimental import pallas as pl
from jax.experimental.pallas import tpu as pltpu
from jax.experimental.pallas import tpu_sc as plsc

B = 1024
D = 16
NUM_ENT = 1000000
NUM_TYPE = 1000
NUM_CORES = 2
NUM_SUBCORES = 16
NW = NUM_CORES * NUM_SUBCORES
B_PER_W = B // NW
SLAB = 128
GRP = 16


def _sc_gather(ent, ent_type, ee_t, te_t, ep_t, tp_t):
    mesh = plsc.VectorSubcoreMesh(core_axis_name="c", subcore_axis_name="s")
    row_blk_t = jax.ShapeDtypeStruct((NW, B_PER_W, D), jnp.float32)

    @functools.partial(
        pl.kernel,
        mesh=mesh,
        compiler_params=pltpu.CompilerParams(needs_layout_passes=False),
        out_type=[row_blk_t, row_blk_t, row_blk_t, row_blk_t],
        scratch_types=[
            pltpu.VMEM((B_PER_W,), jnp.int32),
            pltpu.VMEM((B_PER_W,), jnp.int32),
            pltpu.VMEM((GRP, D, SLAB), jnp.float32),
            pltpu.VMEM((GRP, D, SLAB), jnp.float32),
            pltpu.VMEM((D, NUM_TYPE), jnp.float32),
            pltpu.VMEM((D, NUM_TYPE), jnp.float32),
            pltpu.VMEM((B_PER_W, D), jnp.float32),
            pltpu.VMEM((B_PER_W, D), jnp.float32),
            pltpu.VMEM((B_PER_W, D), jnp.float32),
            pltpu.VMEM((B_PER_W, D), jnp.float32),
            pltpu.SemaphoreType.DMA,
            pltpu.SemaphoreType.DMA,
            pltpu.SemaphoreType.DMA,
            pltpu.SemaphoreType.DMA,
        ],
    )
    def gather_kernel(ent_hbm, etype_hbm, ee_hbm, te_hbm, ep_hbm, tp_hbm,
                      e_out, t_out, pe_out, pt_out,
                      idx_e, idx_t, slab0, slab1, ty_e, ty_p,
                      e_v, t_v, pe_v, pt_v, sem0, sem1, sem_ty, sem_out):
        wid = lax.axis_index("s") * NUM_CORES + lax.axis_index("c")
        base = wid * B_PER_W
        sl = pl.ds(base, B_PER_W)
        ci_e = pltpu.async_copy(ent_hbm.at[sl], idx_e, sem_out)
        ci_t = pltpu.async_copy(etype_hbm.at[sl], idx_t, sem_out)
        ct_e = pltpu.async_copy(te_hbm, ty_e, sem_ty)
        ct_p = pltpu.async_copy(tp_hbm, ty_p, sem_ty)
        ci_e.wait()
        ci_t.wait()
        grp16 = lax.iota(jnp.int32, GRP)
        rings = (slab0, slab1)
        sems = (sem0, sem1)

        def fire(table_hbm, g, ring):
            ve = idx_e[pl.ds(g * GRP, GRP)]
            cps = []
            for k in range(GRP):
                ie = ve[k]
                col = pl.multiple_of(ie - lax.rem(ie, SLAB), SLAB)
                cps.append(pltpu.async_copy(
                    table_hbm.at[:, pl.ds(col, SLAB)],
                    rings[ring].at[k], sems[ring]))
            return cps

        def extract(dst_v, g, ring):
            ve = idx_e[pl.ds(g * GRP, GRP)]
            lane_v = ve % SLAB
            row_v = grp16 + g * GRP
            for d in range(D):
                d_v = jnp.full((GRP,), d, jnp.int32)
                vec = plsc.load_gather(rings[ring], [grp16, d_v, lane_v])
                plsc.store_scatter(dst_v, [row_v, d_v], vec)

        f0 = fire(ee_hbm, 0, 0)
        f1 = fire(ee_hbm, 1, 1)
        ct_e.wait()
        ct_p.wait()
        for g2 in range(B_PER_W // GRP):
            vt = idx_t[pl.ds(g2 * GRP, GRP)]
            row_v2 = grp16 + g2 * GRP
            for d in range(D):
                d_v = jnp.full((GRP,), d, jnp.int32)
                vec_t = plsc.load_gather(ty_e, [d_v, vt])
                plsc.store_scatter(t_v, [row_v2, d_v], vec_t)
                vec_p = plsc.load_gather(ty_p, [d_v, vt])
                plsc.store_scatter(pt_v, [row_v2, d_v], vec_p)
        co_t = pltpu.async_copy(t_v, t_out.at[wid], sem_out)
        co_pt = pltpu.async_copy(pt_v, pt_out.at[wid], sem_out)
        for c in f0:
            c.wait()
        extract(e_v, 0, 0)
        f2 = fire(ep_hbm, 0, 0)
        for c in f1:
            c.wait()
        extract(e_v, 1, 1)
        co_e = pltpu.async_copy(e_v, e_out.at[wid], sem_out)
        f3 = fire(ep_hbm, 1, 1)
        for c in f2:
            c.wait()
        extract(pe_v, 0, 0)
        for c in f3:
            c.wait()
        extract(pe_v, 1, 1)
        co_pe = pltpu.async_copy(pe_v, pe_out.at[wid], sem_out)
        co_t.wait()
        co_pt.wait()
        co_e.wait()
        co_pe.wait()

    return gather_kernel(ent, ent_type, ee_t, te_t, ep_t, tp_t)


BI = 512


def _score_body(e_ref, t_ref, pe_ref, pt_ref, out_ref):
    bf = jnp.bfloat16
    e = e_ref[...].reshape(BI, D).astype(bf)
    t = t_ref[...].reshape(BI, D).astype(bf)
    pe = pe_ref[...].reshape(B, D).astype(bf)
    pt = pt_ref[...].reshape(B, D).astype(bf)
    eye = jax.lax.broadcasted_iota(jnp.int32, (D, D), 0)
    eye = (eye == jax.lax.broadcasted_iota(jnp.int32, (D, D), 1)).astype(bf)
    pet = jax.lax.dot_general(eye, pe, (((1,), (1,)), ((), ())),
                              preferred_element_type=jnp.float32
                              ).astype(bf)
    ptt = jax.lax.dot_general(eye, pt, (((1,), (1,)), ((), ())),
                              preferred_element_type=jnp.float32
                              ).astype(bf)
    a = jax.lax.dot_general(e, pet, (((1,), (0,)), ((), ())),
                            preferred_element_type=jnp.float32
                            ).astype(jnp.bfloat16)
    c = jax.lax.dot_general(t, ptt, (((1,), (0,)), ((), ())),
                            preferred_element_type=jnp.float32
                            ).astype(jnp.bfloat16)
    diff = e - t
    out = jnp.zeros((BI, B), jnp.float32)
    for h in range(4):
        acc = jnp.zeros((BI, B), jnp.bfloat16)
        for d in range(4 * h, 4 * h + 4):
            term = diff[:, d:d + 1] + a * pet[d:d + 1, :] - c * ptt[d:d + 1, :]
            acc = acc + jnp.abs(term)
        out = out + acc.astype(jnp.float32)
    out_ref[...] = out


WPB = BI // B_PER_W


def _tc_score(e_b, t_b, pe_b, pt_b):
    return pl.pallas_call(
        _score_body,
        grid=(B // BI,),
        in_specs=[
            pl.BlockSpec((WPB, B_PER_W, D), lambda i: (i, 0, 0)),
            pl.BlockSpec((WPB, B_PER_W, D), lambda i: (i, 0, 0)),
            pl.BlockSpec((NW, B_PER_W, D), lambda i: (0, 0, 0)),
            pl.BlockSpec((NW, B_PER_W, D), lambda i: (0, 0, 0)),
        ],
        out_specs=pl.BlockSpec((BI, B), lambda i: (i, 0)),
        out_shape=jax.ShapeDtypeStruct((B, B), jnp.float32),
    )(e_b, t_b, pe_b, pt_b)


def kernel(ent, ent_type, ent_emb, type_emb, ent_proj, type_proj):
    e_b, t_b, pe_b, pt_b = _sc_gather(
        ent, ent_type,
        ent_emb.T, type_emb.T, ent_proj.T, type_proj.T)
    return _tc_score(e_b, t_b, pe_b, pt_b)
